# Initial kernel scaffold; baseline (speedup 1.0000x reference)
import jax
import jax.numpy as jnp
from jax.experimental import pallas as pl


def kernel(z, embedding_weight):
    raise NotImplementedError("write your pallas kernel here")


# speedup vs baseline: 1.2099x; 1.2099x over previous
import functools

import jax
import jax.numpy as jnp
from jax import lax
from jax.experimental import pallas as pl
from jax.experimental.pallas import tpu as pltpu
from jax.experimental.pallas import tpu_sc as plsc

_NUM_EMBED = 8192
_EMBED_DIM = 256
_BETA = 0.25
_EPS = 1e-12

_ROW_BLOCK = 256
_K_CHUNK = 1024

_SC_CORES = 2
_SC_SUBCORES = 16
_SC_WORKERS = _SC_CORES * _SC_SUBCORES
_GATHER_CHUNK = 128


def _normalize_cb_body(w_ref, out_ref):
    w = w_ref[...]
    n = jnp.sqrt(jnp.sum(w * w, axis=1, keepdims=True))
    out_ref[...] = w / jnp.maximum(n, _EPS)


def _argmax_body(z_ref, ncb_ref, idx_ref):
    z = z_ref[...]
    zn = jnp.sqrt(jnp.sum(z * z, axis=1, keepdims=True))
    nz = (z / jnp.maximum(zn, _EPS)).astype(jnp.bfloat16)
    run_m = jnp.full((_ROW_BLOCK,), -jnp.inf, dtype=jnp.float32)
    run_i = jnp.zeros((_ROW_BLOCK,), dtype=jnp.int32)
    for kc in range(_NUM_EMBED // _K_CHUNK):
        cb = ncb_ref[pl.ds(kc * _K_CHUNK, _K_CHUNK), :].astype(jnp.bfloat16)
        d = lax.dot_general(nz, cb, (((1,), (1,)), ((), ())),
                            preferred_element_type=jnp.float32)
        m = jnp.max(d, axis=1)
        ii = lax.broadcasted_iota(jnp.int32, d.shape, 1)
        a = jnp.min(jnp.where(d == m[:, None], ii, _K_CHUNK),
                    axis=1) + kc * _K_CHUNK
        upd = m > run_m
        run_i = jnp.where(upd, a, run_i)
        run_m = jnp.where(upd, m, run_m)
    idx_ref[0, 0, :] = run_i


def _st_loss_body(z_ref, zq_ref, st_ref, acc_ref):
    i = pl.program_id(0)
    z = z_ref[...]
    diff = zq_ref[...] - z
    st_ref[...] = z + diff
    s = jnp.sum(diff * diff).reshape(1, 1)

    @pl.when(i == 0)
    def _init():
        acc_ref[...] = s

    @pl.when(i > 0)
    def _acc():
        acc_ref[...] += s


def _gather_body(table_hbm, idx_hbm, out_hbm, idx_v, rows_v, sem):
    wid = lax.axis_index("s") * _SC_CORES + lax.axis_index("c")
    per_w = (16 * 1024) // _SC_WORKERS
    for c in range(per_w // _GATHER_CHUNK):
        base = wid * per_w + c * _GATHER_CHUNK
        pltpu.sync_copy(idx_hbm.at[pl.ds(base, _GATHER_CHUNK)], idx_v)
        pltpu.async_copy(table_hbm.at[idx_v], rows_v, sem).wait()
        pltpu.sync_copy(rows_v, out_hbm.at[pl.ds(base, _GATHER_CHUNK)])


def kernel(z, embedding_weight):
    b, hw, dim = z.shape
    n = b * hw
    z_flat = z.reshape(n, dim)
    num_blocks = n // _ROW_BLOCK

    ncb = pl.pallas_call(
        _normalize_cb_body,
        out_shape=jax.ShapeDtypeStruct((_NUM_EMBED, _EMBED_DIM), jnp.float32),
    )(embedding_weight)

    idx3 = pl.pallas_call(
        _argmax_body,
        grid=(num_blocks,),
        in_specs=[
            pl.BlockSpec((_ROW_BLOCK, _EMBED_DIM), lambda i: (i, 0)),
            pl.BlockSpec((_NUM_EMBED, _EMBED_DIM), lambda i: (0, 0)),
        ],
        out_specs=pl.BlockSpec((1, 1, _ROW_BLOCK), lambda i: (i, 0, 0)),
        out_shape=jax.ShapeDtypeStruct((num_blocks, 1, _ROW_BLOCK), jnp.int32),
    )(z_flat, ncb)
    indices = idx3.reshape(n)

    mesh = plsc.VectorSubcoreMesh(
        core_axis_name="c", subcore_axis_name="s",
        num_cores=_SC_CORES, num_subcores=_SC_SUBCORES)
    gather = pl.kernel(
        _gather_body,
        out_type=jax.ShapeDtypeStruct((n, _EMBED_DIM), jnp.float32),
        mesh=mesh,
        scratch_types=[
            pltpu.VMEM((_GATHER_CHUNK,), jnp.int32),
            pltpu.VMEM((_GATHER_CHUNK, _EMBED_DIM), jnp.float32),
            pltpu.SemaphoreType.DMA,
        ],
    )
    z_q = gather(embedding_weight, indices)

    z_q_st, acc = pl.pallas_call(
        _st_loss_body,
        grid=(num_blocks,),
        in_specs=[
            pl.BlockSpec((_ROW_BLOCK, _EMBED_DIM), lambda i: (i, 0)),
            pl.BlockSpec((_ROW_BLOCK, _EMBED_DIM), lambda i: (i, 0)),
        ],
        out_specs=[
            pl.BlockSpec((_ROW_BLOCK, _EMBED_DIM), lambda i: (i, 0)),
            pl.BlockSpec((1, 1), lambda i: (0, 0)),
        ],
        out_shape=[
            jax.ShapeDtypeStruct((n, _EMBED_DIM), jnp.float32),
            jax.ShapeDtypeStruct((1, 1), jnp.float32),
        ],
    )(z_flat, z_q)

    mean = acc[0, 0] / jnp.float32(n * _EMBED_DIM)
    loss = _BETA * mean + mean
    return z_q_st.reshape(z.shape), loss, indices
